# baseline (speedup 1.0000x reference)
import functools
import math

import jax
import jax.numpy as jnp
from jax import lax
from jax.experimental import pallas as pl
from jax.experimental.pallas import tpu as pltpu
from jax.experimental.pallas import tpu_sc as plsc

VOCAB = 1_000_000
D = 128
B, L = 4096, 200
N = B * L
NC, NS = 2, 16
NW = NC * NS
PER_W = N // NW
G = 128
GROUPS = PER_W // G
NBUF = 5
P = 3
SCALE = float(math.sqrt(128.0))

_mesh = plsc.VectorSubcoreMesh(core_axis_name="c", subcore_axis_name="s")


@functools.partial(
    pl.kernel,
    mesh=_mesh,
    out_type=jax.ShapeDtypeStruct((N, D), jnp.float32),
    scratch_types=[
        pltpu.VMEM((GROUPS, G), jnp.int32),
        pltpu.VMEM((NBUF, G, D), jnp.float32),
        pltpu.SemaphoreType.DMA,
        pltpu.SemaphoreType.DMA,
        pltpu.SemaphoreType.DMA,
        pltpu.SemaphoreType.DMA,
        pltpu.SemaphoreType.DMA,
        pltpu.SemaphoreType.DMA,
        pltpu.SemaphoreType.DMA,
        pltpu.SemaphoreType.DMA,
        pltpu.SemaphoreType.DMA,
        pltpu.SemaphoreType.DMA,
    ],
)
def _embed_sc(idx_hbm, table_hbm, out_hbm, idx_v, rows_v,
              g0, g1, g2, g3, g4, s0, s1, s2, s3, s4):
    gsems = [g0, g1, g2, g3, g4]
    ssems = [s0, s1, s2, s3, s4]
    wid = lax.axis_index("s") * NC + lax.axis_index("c")
    row_base = wid * GROUPS
    pltpu.sync_copy(idx_hbm.at[pl.ds(row_base, GROUPS)], idx_v)

    def start_gather(g, b):
        pltpu.async_copy(table_hbm.at[idx_v.at[g]], rows_v.at[b], gsems[b])

    def wait_gather(g, b):
        pltpu.make_async_copy(table_hbm.at[idx_v.at[g]], rows_v.at[b],
                              gsems[b]).wait()

    def start_scatter(g, b):
        pltpu.async_copy(rows_v.at[b],
                         out_hbm.at[pl.ds((row_base + g) * G, G)],
                         ssems[b])

    def wait_scatter(g, b):
        pltpu.make_async_copy(rows_v.at[b],
                              out_hbm.at[pl.ds((row_base + g) * G, G)],
                              ssems[b]).wait()

    def scale(b):
        def scale_rows(r2, c2):
            for dr in range(2):
                for c in range(D // 16):
                    sl = pl.ds(c * 16, 16)
                    rows_v[b, r2 * 2 + dr, sl] = rows_v[b, r2 * 2 + dr, sl] * SCALE
            return c2

        lax.fori_loop(0, G // 2, scale_rows, 0)

    def step(s):
        if s < GROUPS:
            wait_gather(s, s % NBUF)
        if 2 <= s <= GROUPS + 1:
            wait_scatter(s - 2, (s - 2) % NBUF)
        if s < GROUPS:
            scale(s % NBUF)
        if 1 <= s <= GROUPS:
            start_scatter(s - 1, (s - 1) % NBUF)
        if s + P < GROUPS:
            start_gather(s + P, (s + P) % NBUF)

    for g in range(P):
        start_gather(g, g % NBUF)
    for s in range(NBUF):
        step(s)

    def outer(k, carry):
        for b in range(NBUF):
            wait_gather(k * NBUF + b, b)
            wait_scatter(k * NBUF + b - 2, (b - 2) % NBUF)
            scale(b)
            start_scatter(k * NBUF + b - 1, (b - 1) % NBUF)
            start_gather(k * NBUF + b + P, (b + P) % NBUF)
        return carry

    lax.fori_loop(1, GROUPS // NBUF - 1, outer, 0)

    for s in range(GROUPS - NBUF, GROUPS + 2):
        step(s)
    plsc.subcore_barrier()


def kernel(x, input_embedding):
    idx = x.astype(jnp.int32).reshape(N // G, G)
    out = _embed_sc(idx, input_embedding)
    return out.reshape(B, L, D)
